# baseline (device time: 22667 ns/iter reference)
import jax
import jax.numpy as jnp
from jax import lax
from jax.experimental import pallas as pl
from jax.experimental.pallas import tpu as pltpu

N_DEV = 8
B, SQ, DM = 2, 256, 512
HQ, DH = 32, 64
H_LOC = HQ // N_DEV
DLOC = H_LOC * DH
ROWS = B * SQ
CH = ROWS // N_DEV
BLK = 64
BF = jnp.bfloat16
F32 = jnp.float32


def kernel(x, Wq, K_ext, V_ext, Wo):
    idx = lax.axis_index("i")
    wq_s = lax.dynamic_slice_in_dim(Wq, idx * DLOC, DLOC, axis=1).astype(BF)
    wo_s = lax.dynamic_slice_in_dim(Wo, idx * DLOC, DLOC, axis=0).astype(BF)
    x2 = (x.reshape(ROWS, DM) * 0.125).astype(BF)
    k2 = K_ext.reshape(ROWS, DLOC).astype(BF)
    v2 = V_ext.reshape(ROWS, DLOC).astype(BF)
    eye = jnp.eye(H_LOC, dtype=F32)
    BO = jnp.repeat(jnp.repeat(eye, BLK, axis=0), DH, axis=1).astype(BF)

    def body(x_ref, wq_ref, k_ref, v_ref, bo_ref, wo_ref, out_ref,
             send_buf, red_ref, p1_buf, p2_buf, kbd_ref, vbd_ref,
             p1_send, p1_recv, p2_send, p2_recv):
        my = lax.axis_index("i")

        barrier_sem = pltpu.get_barrier_semaphore()
        for o in range(1, N_DEV):
            pl.semaphore_signal(
                barrier_sem, inc=1,
                device_id=(lax.rem(my + o, N_DEV),),
                device_id_type=pl.DeviceIdType.MESH,
            )

        wq = wq_ref[...]
        wo = wo_ref[...]
        bo = bo_ref[...]

        k3 = k_ref[...].reshape(N_DEV, BLK, DLOC)
        v3 = v_ref[...].reshape(N_DEV, BLK, DLOC)
        for h in range(H_LOC):
            bo_h = bo[h * BLK:(h + 1) * BLK, :]
            kbd_ref[:, h * BLK:(h + 1) * BLK, :] = k3 * bo_h[None, :, :]
            vbd_ref[:, h * BLK:(h + 1) * BLK, :] = v3 * bo_h[None, :, :]

        def chunk(dd, out_dtype):
            xc = x_ref[pl.ds(dd * CH, CH), :]
            q = jnp.dot(xc, wq, preferred_element_type=F32).astype(BF)
            s = lax.dot_general(
                q, kbd_ref[dd], (((1,), (1,)), ((), ())),
                preferred_element_type=F32)
            w = jnp.exp(s).astype(BF)
            ctx_raw = jnp.dot(w, vbd_ref[dd], preferred_element_type=F32)
            den = jnp.dot(w, bo, preferred_element_type=F32)
            ctx = (ctx_raw / den).astype(BF)
            out = jnp.dot(ctx, wo, preferred_element_type=F32)
            return out.astype(out_dtype)

        p1 = []
        for o in range(1, N_DEV):
            tgt = lax.rem(my + o, N_DEV)
            send_buf[o - 1, :, :] = chunk(tgt, BF)
            if o == 1:
                pl.semaphore_wait(barrier_sem, N_DEV - 1)
            r = pltpu.make_async_remote_copy(
                src_ref=send_buf.at[o - 1],
                dst_ref=p1_buf.at[o - 1],
                send_sem=p1_send.at[o - 1],
                recv_sem=p1_recv.at[o - 1],
                device_id=(tgt,),
                device_id_type=pl.DeviceIdType.MESH,
            )
            r.start()
            p1.append(r)

        red = chunk(my, F32)
        for o, r in enumerate(p1):
            r.wait_recv()
            red = red + p1_buf[o, :, :].astype(F32)
        red_ref[...] = red.astype(BF)
        out_ref[pl.ds(my * CH, CH), :] = red

        p2 = []
        for o in range(1, N_DEV):
            tgt = lax.rem(my + o, N_DEV)
            r = pltpu.make_async_remote_copy(
                src_ref=red_ref,
                dst_ref=p2_buf.at[pl.ds(my * CH, CH), :],
                send_sem=p2_send.at[o - 1],
                recv_sem=p2_recv.at[o - 1],
                device_id=(tgt,),
                device_id_type=pl.DeviceIdType.MESH,
            )
            r.start()
            p2.append(r)
        for o, r in enumerate(p2, start=1):
            r.wait_recv()
            src = lax.rem(my - o + N_DEV, N_DEV)
            rs = pl.ds(src * CH, CH)
            out_ref[rs, :] = p2_buf[rs, :].astype(F32)
        for r in p1:
            r.wait_send()
        for r in p2:
            r.wait_send()

    out2 = pl.pallas_call(
        body,
        out_shape=jax.ShapeDtypeStruct((ROWS, DM), F32),
        in_specs=[pl.BlockSpec(memory_space=pltpu.VMEM)] * 6,
        out_specs=pl.BlockSpec(memory_space=pltpu.VMEM),
        scratch_shapes=[
            pltpu.VMEM((N_DEV - 1, CH, DM), BF),
            pltpu.VMEM((CH, DM), BF),
            pltpu.VMEM((N_DEV - 1, CH, DM), BF),
            pltpu.VMEM((ROWS, DM), BF),
            pltpu.VMEM((N_DEV, H_LOC * BLK, DLOC), BF),
            pltpu.VMEM((N_DEV, H_LOC * BLK, DLOC), BF),
            pltpu.SemaphoreType.DMA((N_DEV - 1,)),
            pltpu.SemaphoreType.DMA((N_DEV - 1,)),
            pltpu.SemaphoreType.DMA((N_DEV - 1,)),
            pltpu.SemaphoreType.DMA((N_DEV - 1,)),
        ],
        compiler_params=pltpu.CompilerParams(collective_id=0),
    )(x2, wq_s, k2, v2, BO, wo_s)
    return out2.reshape(B, SQ, DM)


# device time: 21719 ns/iter; 1.0436x vs baseline; 1.0436x over previous
import jax
import jax.numpy as jnp
from jax import lax
from jax.experimental import pallas as pl
from jax.experimental.pallas import tpu as pltpu

N_DEV = 8
B, SQ, DM = 2, 256, 512
HQ, DH = 32, 64
H_LOC = HQ // N_DEV
DLOC = H_LOC * DH
ROWS = B * SQ
CH = ROWS // N_DEV
BLK = 64
BF = jnp.bfloat16
F32 = jnp.float32


def kernel(x, Wq, K_ext, V_ext, Wo):
    idx = lax.axis_index("i")
    wq_s = lax.dynamic_slice_in_dim(Wq, idx * DLOC, DLOC, axis=1)
    wo_s = lax.dynamic_slice_in_dim(Wo, idx * DLOC, DLOC, axis=0)
    k2 = K_ext.reshape(ROWS, DLOC)
    v2 = V_ext.reshape(ROWS, DLOC)

    def body(x_ref, wq_ref, k_ref, v_ref, wo_ref, out_ref,
             xbf_ref, send_buf, red_ref, p1_buf, p2_buf, kbd_ref, vbd_ref,
             p1_send, p1_recv, p2_send, p2_recv):
        my = lax.axis_index("i")

        barrier_sem = pltpu.get_barrier_semaphore()
        for o in range(1, N_DEV):
            pl.semaphore_signal(
                barrier_sem, inc=1,
                device_id=(lax.rem(my + o, N_DEV),),
                device_id_type=pl.DeviceIdType.MESH,
            )

        xbf_ref[...] = (x_ref[...].reshape(ROWS, DM) * 0.125).astype(BF)
        wq = wq_ref[...].astype(BF)
        wo = wo_ref[...].astype(BF)

        ri = lax.broadcasted_iota(jnp.int32, (DLOC, DLOC), 0) // BLK
        ci = lax.broadcasted_iota(jnp.int32, (DLOC, DLOC), 1) // BLK
        bo = (ri == ci).astype(BF)

        k3 = k_ref[...].astype(BF).reshape(N_DEV, BLK, DLOC)
        v3 = v_ref[...].astype(BF).reshape(N_DEV, BLK, DLOC)
        for h in range(H_LOC):
            bo_h = bo[h * BLK:(h + 1) * BLK, :]
            kbd_ref[:, h * BLK:(h + 1) * BLK, :] = k3 * bo_h[None, :, :]
            vbd_ref[:, h * BLK:(h + 1) * BLK, :] = v3 * bo_h[None, :, :]

        def chunk(dd, out_dtype):
            xc = xbf_ref[pl.ds(dd * CH, CH), :]
            q = jnp.dot(xc, wq, preferred_element_type=F32).astype(BF)
            s = lax.dot_general(
                q, kbd_ref[dd], (((1,), (1,)), ((), ())),
                preferred_element_type=F32)
            w = jnp.exp(s).astype(BF)
            ctx_raw = jnp.dot(w, vbd_ref[dd], preferred_element_type=F32)
            den = jnp.dot(w, bo, preferred_element_type=F32)
            ctx = (ctx_raw / den).astype(BF)
            out = jnp.dot(ctx, wo, preferred_element_type=F32)
            return out.astype(out_dtype)

        def store_out(dd, val):
            out_ref[dd // 4, pl.ds(lax.rem(dd, 4) * CH, CH), :] = val

        p1 = []
        for o in range(1, N_DEV):
            tgt = lax.rem(my + o, N_DEV)
            send_buf[o - 1, :, :] = chunk(tgt, BF)
            if o == 1:
                pl.semaphore_wait(barrier_sem, N_DEV - 1)
            r = pltpu.make_async_remote_copy(
                src_ref=send_buf.at[o - 1],
                dst_ref=p1_buf.at[o - 1],
                send_sem=p1_send.at[o - 1],
                recv_sem=p1_recv.at[o - 1],
                device_id=(tgt,),
                device_id_type=pl.DeviceIdType.MESH,
            )
            r.start()
            p1.append(r)

        red = chunk(my, F32)
        for o, r in enumerate(p1):
            r.wait_recv()
            red = red + p1_buf[o, :, :].astype(F32)
        red_ref[...] = red.astype(BF)
        store_out(my, red)

        p2 = []
        for o in range(1, N_DEV):
            tgt = lax.rem(my + o, N_DEV)
            r = pltpu.make_async_remote_copy(
                src_ref=red_ref,
                dst_ref=p2_buf.at[pl.ds(my * CH, CH), :],
                send_sem=p2_send.at[o - 1],
                recv_sem=p2_recv.at[o - 1],
                device_id=(tgt,),
                device_id_type=pl.DeviceIdType.MESH,
            )
            r.start()
            p2.append(r)
        for o, r in enumerate(p2, start=1):
            r.wait_recv()
            src = lax.rem(my - o + N_DEV, N_DEV)
            store_out(src, p2_buf[pl.ds(src * CH, CH), :].astype(F32))
        for r in p1:
            r.wait_send()
        for r in p2:
            r.wait_send()

    return pl.pallas_call(
        body,
        out_shape=jax.ShapeDtypeStruct((B, SQ, DM), F32),
        in_specs=[pl.BlockSpec(memory_space=pltpu.VMEM)] * 5,
        out_specs=pl.BlockSpec(memory_space=pltpu.VMEM),
        scratch_shapes=[
            pltpu.VMEM((ROWS, DM), BF),
            pltpu.VMEM((N_DEV - 1, CH, DM), BF),
            pltpu.VMEM((CH, DM), BF),
            pltpu.VMEM((N_DEV - 1, CH, DM), BF),
            pltpu.VMEM((ROWS, DM), BF),
            pltpu.VMEM((N_DEV, H_LOC * BLK, DLOC), BF),
            pltpu.VMEM((N_DEV, H_LOC * BLK, DLOC), BF),
            pltpu.SemaphoreType.DMA((N_DEV - 1,)),
            pltpu.SemaphoreType.DMA((N_DEV - 1,)),
            pltpu.SemaphoreType.DMA((N_DEV - 1,)),
            pltpu.SemaphoreType.DMA((N_DEV - 1,)),
        ],
        compiler_params=pltpu.CompilerParams(collective_id=0),
    )(x, wq_s, k2, v2, wo_s)


# device time: 21711 ns/iter; 1.0440x vs baseline; 1.0004x over previous
import jax
import jax.numpy as jnp
from jax import lax
from jax.experimental import pallas as pl
from jax.experimental.pallas import tpu as pltpu

N_DEV = 8
B, SQ, DM = 2, 256, 512
HQ, DH = 32, 64
H_LOC = HQ // N_DEV
DLOC = H_LOC * DH
ROWS = B * SQ
CH = ROWS // N_DEV
BLK = 64
BF = jnp.bfloat16
F32 = jnp.float32


def kernel(x, Wq, K_ext, V_ext, Wo):
    idx = lax.axis_index("i")
    wq_s = lax.dynamic_slice_in_dim(Wq, idx * DLOC, DLOC, axis=1)
    wo_s = lax.dynamic_slice_in_dim(Wo, idx * DLOC, DLOC, axis=0)
    k2 = K_ext.reshape(ROWS, DLOC)
    v2 = V_ext.reshape(ROWS, DLOC)

    def body(x_ref, wq_ref, k_ref, v_ref, wo_ref, out_ref,
             xbf_ref, send_buf, red_ref, p1_buf, p2_buf, kbd_ref, vbd_ref,
             p1_send, p1_recv, p2_send, p2_recv):
        my = lax.axis_index("i")

        barrier_sem = pltpu.get_barrier_semaphore()
        for o in range(1, N_DEV):
            pl.semaphore_signal(
                barrier_sem, inc=1,
                device_id=(lax.rem(my + o, N_DEV),),
                device_id_type=pl.DeviceIdType.MESH,
            )

        xbf_ref[...] = (x_ref[...].reshape(ROWS, DM) * 0.125).astype(BF)
        wq = wq_ref[...].astype(BF)
        wo = wo_ref[...].astype(BF)

        ri = lax.broadcasted_iota(jnp.int32, (DLOC, DLOC), 0) // BLK
        ci = lax.broadcasted_iota(jnp.int32, (DLOC, DLOC), 1) // BLK
        bo = (ri == ci).astype(BF)

        k3 = k_ref[...].astype(BF).reshape(N_DEV, BLK, DLOC)
        v3 = v_ref[...].astype(BF).reshape(N_DEV, BLK, DLOC)
        for h in range(H_LOC):
            bo_h = bo[h * BLK:(h + 1) * BLK, :]
            kbd_ref[:, h * BLK:(h + 1) * BLK, :] = k3 * bo_h[None, :, :]
            vbd_ref[:, h * BLK:(h + 1) * BLK, :] = v3 * bo_h[None, :, :]

        def chunk(dd, out_dtype):
            xc = xbf_ref[pl.ds(dd * CH, CH), :]
            q = jnp.dot(xc, wq, preferred_element_type=F32).astype(BF)
            s = lax.dot_general(
                q, kbd_ref[dd], (((1,), (1,)), ((), ())),
                preferred_element_type=F32)
            w = jnp.exp(s).astype(BF)
            ctx_raw = jnp.dot(w, vbd_ref[dd], preferred_element_type=F32)
            den = jnp.dot(w, bo, preferred_element_type=F32)
            ctx = (ctx_raw / den).astype(BF)
            out = jnp.dot(ctx, wo, preferred_element_type=F32)
            return out.astype(out_dtype)

        def store_out(dd, val):
            out_ref[dd // 4, pl.ds(lax.rem(dd, 4) * CH, CH), :] = val

        p1 = []
        for o in range(1, N_DEV):
            tgt = lax.rem(my + o, N_DEV)
            send_buf[o - 1, :, :] = chunk(tgt, BF)
            if o == 1:
                pl.semaphore_wait(barrier_sem, N_DEV - 1)
            r = pltpu.make_async_remote_copy(
                src_ref=send_buf.at[o - 1],
                dst_ref=p1_buf.at[o - 1],
                send_sem=p1_send.at[o - 1],
                recv_sem=p1_recv.at[o - 1],
                device_id=(tgt,),
                device_id_type=pl.DeviceIdType.MESH,
            )
            r.start()
            p1.append(r)

        red = chunk(my, F32)
        for o, r in enumerate(p1):
            r.wait_recv()
            red = red + p1_buf[o, :, :].astype(F32)
        red_ref[...] = red.astype(BF)

        p2 = []
        for o in range(1, N_DEV):
            tgt = lax.rem(my + o, N_DEV)
            r = pltpu.make_async_remote_copy(
                src_ref=red_ref,
                dst_ref=p2_buf.at[pl.ds(my * CH, CH), :],
                send_sem=p2_send.at[o - 1],
                recv_sem=p2_recv.at[o - 1],
                device_id=(tgt,),
                device_id_type=pl.DeviceIdType.MESH,
            )
            r.start()
            p2.append(r)
        store_out(my, red)
        for o, r in enumerate(p2, start=1):
            r.wait_recv()
            src = lax.rem(my - o + N_DEV, N_DEV)
            store_out(src, p2_buf[pl.ds(src * CH, CH), :].astype(F32))
        for r in p1:
            r.wait_send()
        for r in p2:
            r.wait_send()

    return pl.pallas_call(
        body,
        out_shape=jax.ShapeDtypeStruct((B, SQ, DM), F32),
        in_specs=[pl.BlockSpec(memory_space=pltpu.VMEM)] * 5,
        out_specs=pl.BlockSpec(memory_space=pltpu.VMEM),
        scratch_shapes=[
            pltpu.VMEM((ROWS, DM), BF),
            pltpu.VMEM((N_DEV - 1, CH, DM), BF),
            pltpu.VMEM((CH, DM), BF),
            pltpu.VMEM((N_DEV - 1, CH, DM), BF),
            pltpu.VMEM((ROWS, DM), BF),
            pltpu.VMEM((N_DEV, H_LOC * BLK, DLOC), BF),
            pltpu.VMEM((N_DEV, H_LOC * BLK, DLOC), BF),
            pltpu.SemaphoreType.DMA((N_DEV - 1,)),
            pltpu.SemaphoreType.DMA((N_DEV - 1,)),
            pltpu.SemaphoreType.DMA((N_DEV - 1,)),
            pltpu.SemaphoreType.DMA((N_DEV - 1,)),
        ],
        compiler_params=pltpu.CompilerParams(collective_id=0),
    )(x, wq_s, k2, v2, wo_s)


# device time: 21454 ns/iter; 1.0565x vs baseline; 1.0120x over previous
import jax
import jax.numpy as jnp
from jax import lax
from jax.experimental import pallas as pl
from jax.experimental.pallas import tpu as pltpu

N_DEV = 8
B, SQ, DM = 2, 256, 512
HQ, DH = 32, 64
H_LOC = HQ // N_DEV
DLOC = H_LOC * DH
ROWS = B * SQ
CH = ROWS // N_DEV
BLK = 64
BF = jnp.bfloat16
F32 = jnp.float32


def kernel(x, Wq, K_ext, V_ext, Wo):
    idx = lax.axis_index("i")
    wq_s = lax.dynamic_slice_in_dim(Wq, idx * DLOC, DLOC, axis=1)
    wo_s = lax.dynamic_slice_in_dim(Wo, idx * DLOC, DLOC, axis=0)
    k2 = K_ext.reshape(ROWS, DLOC)
    v2 = V_ext.reshape(ROWS, DLOC)

    def body(x_ref, wq_ref, k_ref, v_ref, wo_ref, out_ref,
             q_ref, send_buf, red_ref, p1_buf, p2_buf, kbd_ref, vbd_ref,
             p1_send, p1_recv, p2_send, p2_recv):
        my = lax.axis_index("i")

        barrier_sem = pltpu.get_barrier_semaphore()
        for o in range(1, N_DEV):
            pl.semaphore_signal(
                barrier_sem, inc=1,
                device_id=(lax.rem(my + o, N_DEV),),
                device_id_type=pl.DeviceIdType.MESH,
            )

        xbf = (x_ref[...].reshape(ROWS, DM) * 0.125).astype(BF)
        wq = wq_ref[...].astype(BF)
        wo = wo_ref[...].astype(BF)
        q_ref[...] = jnp.dot(
            xbf, wq, preferred_element_type=F32).astype(BF)

        ri = lax.broadcasted_iota(jnp.int32, (DLOC, DLOC), 0) // BLK
        ci = lax.broadcasted_iota(jnp.int32, (DLOC, DLOC), 1) // BLK
        bo = (ri == ci).astype(BF)

        k3 = k_ref[...].astype(BF).reshape(N_DEV, BLK, DLOC)
        v3 = v_ref[...].astype(BF).reshape(N_DEV, BLK, DLOC)
        for h in range(H_LOC):
            bo_h = bo[h * BLK:(h + 1) * BLK, :]
            kbd_ref[:, h * BLK:(h + 1) * BLK, :] = k3 * bo_h[None, :, :]
            vbd_ref[:, h * BLK:(h + 1) * BLK, :] = v3 * bo_h[None, :, :]

        def chunk(dd, out_dtype):
            q = q_ref[pl.ds(dd * CH, CH), :]
            s = lax.dot_general(
                q, kbd_ref[dd], (((1,), (1,)), ((), ())),
                preferred_element_type=F32)
            w = jnp.exp(s).astype(BF)
            ctx_raw = jnp.dot(w, vbd_ref[dd], preferred_element_type=F32)
            den = jnp.dot(w, bo, preferred_element_type=F32)
            ctx = (ctx_raw / den).astype(BF)
            out = jnp.dot(ctx, wo, preferred_element_type=F32)
            return out.astype(out_dtype)

        def store_out(dd, val):
            out_ref[dd // 4, pl.ds(lax.rem(dd, 4) * CH, CH), :] = val

        p1 = []
        for o in range(1, N_DEV):
            tgt = lax.rem(my + o, N_DEV)
            send_buf[o - 1, :, :] = chunk(tgt, BF)
            if o == 1:
                pl.semaphore_wait(barrier_sem, N_DEV - 1)
            r = pltpu.make_async_remote_copy(
                src_ref=send_buf.at[o - 1],
                dst_ref=p1_buf.at[o - 1],
                send_sem=p1_send.at[o - 1],
                recv_sem=p1_recv.at[o - 1],
                device_id=(tgt,),
                device_id_type=pl.DeviceIdType.MESH,
            )
            r.start()
            p1.append(r)

        red = chunk(my, F32)
        for o, r in enumerate(p1):
            r.wait_recv()
            red = red + p1_buf[o, :, :].astype(F32)
        red_ref[...] = red.astype(BF)

        p2 = []
        for o in range(1, N_DEV):
            tgt = lax.rem(my + o, N_DEV)
            r = pltpu.make_async_remote_copy(
                src_ref=red_ref,
                dst_ref=p2_buf.at[pl.ds(my * CH, CH), :],
                send_sem=p2_send.at[o - 1],
                recv_sem=p2_recv.at[o - 1],
                device_id=(tgt,),
                device_id_type=pl.DeviceIdType.MESH,
            )
            r.start()
            p2.append(r)
        store_out(my, red)
        for o, r in enumerate(p2, start=1):
            r.wait_recv()
            src = lax.rem(my - o + N_DEV, N_DEV)
            store_out(src, p2_buf[pl.ds(src * CH, CH), :].astype(F32))
        for r in p1:
            r.wait_send()
        for r in p2:
            r.wait_send()

    return pl.pallas_call(
        body,
        out_shape=jax.ShapeDtypeStruct((B, SQ, DM), F32),
        in_specs=[pl.BlockSpec(memory_space=pltpu.VMEM)] * 5,
        out_specs=pl.BlockSpec(memory_space=pltpu.VMEM),
        scratch_shapes=[
            pltpu.VMEM((ROWS, DLOC), BF),
            pltpu.VMEM((N_DEV - 1, CH, DM), BF),
            pltpu.VMEM((CH, DM), BF),
            pltpu.VMEM((N_DEV - 1, CH, DM), BF),
            pltpu.VMEM((ROWS, DM), BF),
            pltpu.VMEM((N_DEV, H_LOC * BLK, DLOC), BF),
            pltpu.VMEM((N_DEV, H_LOC * BLK, DLOC), BF),
            pltpu.SemaphoreType.DMA((N_DEV - 1,)),
            pltpu.SemaphoreType.DMA((N_DEV - 1,)),
            pltpu.SemaphoreType.DMA((N_DEV - 1,)),
            pltpu.SemaphoreType.DMA((N_DEV - 1,)),
        ],
        compiler_params=pltpu.CompilerParams(collective_id=0),
    )(x, wq_s, k2, v2, wo_s)
